# baseline (speedup 1.0000x reference)
import functools

import jax
import jax.numpy as jnp
from jax import lax
from jax.experimental import pallas as pl
from jax.experimental.pallas import tpu as pltpu
from jax.experimental.pallas import tpu_sc as plsc

N_USERS = 1000000
N_ITEMS = 100000
HIDDEN = 64
BATCH = 16384
FEAT_BITS = 17

NC = 2
NS = 16
NW = NC * NS
BPW = BATCH // NW
CHUNK = 128
NCHUNK = BPW // CHUNK
NBUF = 4

U1ROWS = (N_USERS + 127) // 128
I1ROWS = (N_ITEMS + 127) // 128


def _sc_gather(W2u2, W2i2, W1up, W1ip, uh, ih, uw, iw):
    mesh = plsc.VectorSubcoreMesh(core_axis_name="c", subcore_axis_name="s")
    row_t = jax.ShapeDtypeStruct((BATCH, 128), jnp.float32)
    out_type = (row_t, row_t, row_t, row_t)

    @functools.partial(
        pl.kernel,
        mesh=mesh,
        out_type=out_type,
        scratch_types=[
            pltpu.VMEM((BPW,), jnp.int32),
            pltpu.VMEM((BPW,), jnp.int32),
            pltpu.VMEM((BPW,), jnp.int32),
            pltpu.VMEM((BPW,), jnp.int32),
        ] + [pltpu.VMEM((CHUNK, 128), jnp.float32)] * NBUF
          + [pltpu.SemaphoreType.DMA] * NBUF
          + [pltpu.SemaphoreType.DMA] * NBUF,
    )
    def k(w2u_hbm, w2i_hbm, w1u_hbm, w1i_hbm, uh_hbm, ih_hbm, uw_hbm, iw_hbm,
          u2_hbm, i2_hbm, g1u_hbm, g1i_hbm,
          uh_v, ih_v, uw_v, iw_v, b0, b1, b2, b3,
          g0, g1, g2, g3, w0, w1, w2, w3):
        wid = lax.axis_index("s") * NC + lax.axis_index("c")
        base = wid * BPW
        pltpu.sync_copy(uh_hbm.at[pl.ds(base, BPW)], uh_v)
        pltpu.sync_copy(ih_hbm.at[pl.ds(base, BPW)], ih_v)
        pltpu.sync_copy(uw_hbm.at[pl.ds(base, BPW)], uw_v)
        pltpu.sync_copy(iw_hbm.at[pl.ds(base, BPW)], iw_v)

        bufs = (b0, b1, b2, b3)
        gsems = (g0, g1, g2, g3)
        wsems = (w0, w1, w2, w3)
        streams = (
            (w2u_hbm, uh_v, u2_hbm),
            (w2i_hbm, ih_v, i2_hbm),
            (w1u_hbm, uw_v, g1u_hbm),
            (w1i_hbm, iw_v, g1i_hbm),
        )
        descs = [(streams[t], c) for c in range(NCHUNK) for t in range(4)]
        nd = len(descs)

        def fire_gather(kk, b):
            (tbl, idxr, _), c = descs[kk]
            return pltpu.async_copy(
                tbl.at[idxr.at[pl.ds(c * CHUNK, CHUNK)]], bufs[b], gsems[b])

        def fire_write(kk, b):
            (_, _, outr), c = descs[kk]
            return pltpu.async_copy(
                bufs[b], outr.at[pl.ds(base + c * CHUNK, CHUNK)], wsems[b])

        gc = [None] * NBUF
        wc = [None] * NBUF
        for kk in range(NBUF):
            gc[kk] = fire_gather(kk, kk)
        for kk in range(nd):
            b = kk % NBUF
            gc[b].wait()
            wc[b] = fire_write(kk, b)
            if kk + NBUF < nd:
                wc[b].wait()
                gc[b] = fire_gather(kk + NBUF, b)
        for kk in range(nd - NBUF, nd):
            wc[kk % NBUF].wait()

    return k(W2u2, W2i2, W1up, W1ip, uh, ih, uw, iw)


def _tc_body(ui_ref, ii_ref, f0_ref, f1_ref, u2_ref, i2_ref, g1u_ref, g1i_ref,
             w2f0_ref, w2f1_ref, w1f_ref, bias_ref, out_ref):
    j = lax.broadcasted_iota(jnp.int32, (1, 128), 1)
    mask = jnp.where(j < FEAT_BITS,
                     jnp.left_shift(1, jnp.maximum(FEAT_BITS - 1 - j, 0)), 0)
    bits0 = (jnp.bitwise_and(f0_ref[...], mask) != 0).astype(jnp.float32)
    bits1 = (jnp.bitwise_and(f1_ref[...], mask) != 0).astype(jnp.float32)
    s0 = jnp.sum(bits0, axis=1, keepdims=True)
    s1 = jnp.sum(bits1, axis=1, keepdims=True)

    w1f = w1f_ref[...]
    fo0 = jnp.sum(bits0 * w1f[0:1, :], axis=1, keepdims=True) / s0
    fo1 = jnp.sum(bits1 * w1f[1:2, :], axis=1, keepdims=True) / s1

    e0 = jnp.dot(bits0, w2f0_ref[...],
                 preferred_element_type=jnp.float32,
                 precision=lax.Precision.HIGHEST) / s0
    e1 = jnp.dot(bits1, w2f1_ref[...],
                 preferred_element_type=jnp.float32,
                 precision=lax.Precision.HIGHEST) / s1

    ui = ui_ref[...]
    ii = ii_ref[...]
    u2row = u2_ref[...]
    i2row = i2_ref[...]
    u2 = jnp.where(jnp.bitwise_and(ui, 1) == 0, u2row[:, :64], u2row[:, 64:])
    i2 = jnp.where(jnp.bitwise_and(ii, 1) == 0, i2row[:, :64], i2row[:, 64:])

    w1u = jnp.sum(g1u_ref[...] * (jnp.bitwise_and(ui, 127) == j),
                  axis=1, keepdims=True)
    w1i = jnp.sum(g1i_ref[...] * (jnp.bitwise_and(ii, 127) == j),
                  axis=1, keepdims=True)

    ssum = u2 + i2 + e0 + e1
    diff = ssum * ssum - (u2 * u2 + i2 * i2 + e0 * e0 + e1 * e1)
    second = 0.5 * jnp.sum(diff, axis=1, keepdims=True)

    out_ref[...] = bias_ref[0, 0] + w1u + w1i + fo0 + fo1 + second


BB = 2048


def _tc_combine(ui, ii, f0, f1, u2, i2, g1u, g1i, W2f0p, W2f1p, w1f, bias2):
    grid = (BATCH // BB,)
    bspec = lambda bs: pl.BlockSpec(bs, lambda i: (i, 0))
    wspec = lambda bs: pl.BlockSpec(bs, lambda i: (0, 0))
    return pl.pallas_call(
        _tc_body,
        grid=grid,
        in_specs=[
            bspec((BB, 1)), bspec((BB, 1)), bspec((BB, 1)), bspec((BB, 1)),
            bspec((BB, 128)), bspec((BB, 128)),
            bspec((BB, 128)), bspec((BB, 128)),
            wspec((128, HIDDEN)), wspec((128, HIDDEN)),
            wspec((2, 128)), wspec((1, 1)),
        ],
        out_specs=bspec((BB, 1)),
        out_shape=jax.ShapeDtypeStruct((BATCH, 1), jnp.float32),
    )(ui, ii, f0, f1, u2, i2, g1u, g1i, W2f0p, W2f1p, w1f, bias2)


def kernel(x, bias, W1u, W1i, W1f0, W1f1, W2u, W2i, W2f0, W2f1):
    uidx = x[:, 0]
    iidx = x[:, 1]
    uh = uidx // 2
    ih = iidx // 2
    uw = uidx // 128
    iw = iidx // 128

    W2u2 = W2u.reshape(N_USERS // 2, 128)
    W2i2 = W2i.reshape(N_ITEMS // 2, 128)
    W1up = jnp.concatenate(
        [W1u.reshape(-1), jnp.zeros((U1ROWS * 128 - N_USERS,), jnp.float32)]
    ).reshape(U1ROWS, 128)
    W1ip = jnp.concatenate(
        [W1i.reshape(-1), jnp.zeros((I1ROWS * 128 - N_ITEMS,), jnp.float32)]
    ).reshape(I1ROWS, 128)

    u2, i2, g1u, g1i = _sc_gather(W2u2, W2i2, W1up, W1ip, uh, ih, uw, iw)

    pad = jnp.zeros((128 - FEAT_BITS, HIDDEN), jnp.float32)
    W2f0p = jnp.concatenate([W2f0, pad], axis=0)
    W2f1p = jnp.concatenate([W2f1, pad], axis=0)
    wpad = jnp.zeros((1, 128 - FEAT_BITS), jnp.float32)
    w1f = jnp.concatenate([
        jnp.concatenate([W1f0.T, wpad], axis=1),
        jnp.concatenate([W1f1.T, wpad], axis=1),
    ], axis=0)

    out = _tc_combine(
        x[:, 0:1], x[:, 1:2], x[:, 2:3], x[:, 3:4],
        u2, i2, g1u, g1i, W2f0p, W2f1p, w1f, bias.reshape(1, 1),
    )
    return out[:, 0]
